# baseline (speedup 1.0000x reference)
import functools

import jax
import jax.numpy as jnp
from jax import lax
from jax.experimental import pallas as pl
from jax.experimental.pallas import tpu as pltpu
from jax.experimental.pallas import tpu_sc as plsc

NPAD = 10240
D = 128
CH = 80
NCH = 125
CH2 = 128
NCH2 = 80
EPAD = 32 * NCH2 * CH2
ROWS_PER_TILE = 640

_mesh = plsc.VectorSubcoreMesh(core_axis_name="c", subcore_axis_name="s")


@functools.partial(
    pl.kernel,
    out_type=jax.ShapeDtypeStruct((2, NPAD), jnp.float32),
    mesh=_mesh,
    scratch_types=[
        pltpu.VMEM((NCH, CH), jnp.int32),
        pltpu.VMEM((CH,), jnp.float32),
        pltpu.VMEM_SHARED((NPAD,), jnp.float32),
    ],
)
def _deg_kernel(col_ref, ones_ref, z1_ref, out_ref, col_v, ones_v, deg_sh):
    cid = lax.axis_index("c")
    sid = lax.axis_index("s")
    pltpu.sync_copy(ones_ref, ones_v)
    pltpu.sync_copy(col_ref.at[cid * 16 + sid], col_v)
    sl = pl.ds(sid * ROWS_PER_TILE, ROWS_PER_TILE)
    pltpu.sync_copy(z1_ref.at[sl], deg_sh.at[sl])
    plsc.subcore_barrier()

    def body(i, carry):
        pltpu.sync_copy(ones_v, deg_sh.at[col_v.at[i]], add=True)
        return carry

    lax.fori_loop(0, NCH, body, 0)
    plsc.subcore_barrier()
    pltpu.sync_copy(deg_sh.at[sl], out_ref.at[cid, sl])


@functools.partial(
    pl.kernel,
    out_type=jax.ShapeDtypeStruct((2, NPAD, D), jnp.float32),
    mesh=_mesh,
    scratch_types=[
        pltpu.VMEM((CH2,), jnp.int32),
        pltpu.VMEM((CH2,), jnp.int32),
        pltpu.VMEM((CH2,), jnp.int32),
        pltpu.VMEM((CH2,), jnp.int32),
        pltpu.VMEM((CH2, D), jnp.float32),
        pltpu.VMEM((CH2, D), jnp.float32),
        pltpu.VMEM_SHARED((NPAD, D), jnp.float32),
        pltpu.SemaphoreType.DMA,
        pltpu.SemaphoreType.DMA,
        pltpu.SemaphoreType.DMA,
        pltpu.SemaphoreType.DMA,
        pltpu.SemaphoreType.DMA,
        pltpu.SemaphoreType.DMA,
        pltpu.SemaphoreType.DMA,
        pltpu.SemaphoreType.DMA,
    ],
)
def _hop_kernel(row_ref, col_ref, g_ref, z2_ref, out_ref,
                ridx0, ridx1, cidx0, cidx1, rows0, rows1, acc_sh,
                gi0r, gi1r, gi0c, gi1c, gg0, gg1, gs0, gs1):
    cid = lax.axis_index("c")
    sid = lax.axis_index("s")
    wid = cid * 16 + sid
    sl = pl.ds(sid * ROWS_PER_TILE, ROWS_PER_TILE)
    pltpu.sync_copy(z2_ref.at[sl], acc_sh.at[sl])
    plsc.subcore_barrier()

    pltpu.async_copy(row_ref.at[wid, 0], ridx0, gi0r)
    pltpu.async_copy(col_ref.at[wid, 0], cidx0, gi0c)
    pltpu.async_copy(row_ref.at[wid, 1], ridx1, gi1r)
    pltpu.async_copy(col_ref.at[wid, 1], cidx1, gi1c)
    pltpu.make_async_copy(row_ref.at[wid, 0], ridx0, gi0r).wait()
    pltpu.async_copy(g_ref.at[ridx0], rows0, gg0)
    pltpu.make_async_copy(row_ref.at[wid, 1], ridx1, gi1r).wait()
    pltpu.async_copy(g_ref.at[ridx1], rows1, gg1)

    def body(i, carry):
        c0 = 2 * i
        c1 = c0 + 1
        pltpu.make_async_copy(g_ref.at[ridx0], rows0, gg0).wait()
        pltpu.make_async_copy(col_ref.at[wid, c0], cidx0, gi0c).wait()
        pltpu.async_copy(rows0, acc_sh.at[cidx0], gs0, add=True)
        pltpu.async_copy(row_ref.at[wid, c0 + 2], ridx0, gi0r)
        pltpu.make_async_copy(g_ref.at[ridx1], rows1, gg1).wait()
        pltpu.make_async_copy(col_ref.at[wid, c1], cidx1, gi1c).wait()
        pltpu.async_copy(rows1, acc_sh.at[cidx1], gs1, add=True)
        pltpu.async_copy(row_ref.at[wid, c1 + 2], ridx1, gi1r)
        pltpu.make_async_copy(rows0, acc_sh.at[cidx0], gs0).wait()
        pltpu.async_copy(col_ref.at[wid, c0 + 2], cidx0, gi0c)
        pltpu.make_async_copy(row_ref.at[wid, c0 + 2], ridx0, gi0r).wait()
        pltpu.async_copy(g_ref.at[ridx0], rows0, gg0)
        pltpu.make_async_copy(rows1, acc_sh.at[cidx1], gs1).wait()
        pltpu.async_copy(col_ref.at[wid, c1 + 2], cidx1, gi1c)
        pltpu.make_async_copy(row_ref.at[wid, c1 + 2], ridx1, gi1r).wait()
        pltpu.async_copy(g_ref.at[ridx1], rows1, gg1)
        return carry

    lax.fori_loop(0, NCH2 // 2 - 1, body, 0)
    cl0 = NCH2 - 2
    cl1 = NCH2 - 1
    pltpu.make_async_copy(g_ref.at[ridx0], rows0, gg0).wait()
    pltpu.make_async_copy(col_ref.at[wid, cl0], cidx0, gi0c).wait()
    pltpu.async_copy(rows0, acc_sh.at[cidx0], gs0, add=True)
    pltpu.make_async_copy(g_ref.at[ridx1], rows1, gg1).wait()
    pltpu.make_async_copy(col_ref.at[wid, cl1], cidx1, gi1c).wait()
    pltpu.async_copy(rows1, acc_sh.at[cidx1], gs1, add=True)
    pltpu.make_async_copy(rows0, acc_sh.at[cidx0], gs0).wait()
    pltpu.make_async_copy(rows1, acc_sh.at[cidx1], gs1).wait()
    plsc.subcore_barrier()
    pltpu.sync_copy(acc_sh.at[sl], out_ref.at[cid, sl])


_BN = 512
_GRID = NPAD // _BN


def _tc1_body(d_ref, x_ref, o_ref):
    cnt = d_ref[0, :] + d_ref[1, :] + 1.0
    dis = lax.rsqrt(cnt)
    o_ref[...] = x_ref[...] * dis[:, None]


def _tc2_body(d_ref, s_ref, g_ref, o_ref):
    cnt = d_ref[0, :] + d_ref[1, :] + 1.0
    inv = 1.0 / cnt
    o_ref[...] = (s_ref[0] + s_ref[1] + g_ref[...]) * inv[:, None]


def _tc3_body(d_ref, x_ref, g2_ref, s2_ref, w0_ref, b0_ref, w1_ref, b1_ref,
              w2_ref, b2_ref, wf_ref, bf_ref, o_ref):
    cnt = d_ref[0, :] + d_ref[1, :] + 1.0
    dis = lax.rsqrt(cnt)
    sq = jnp.sqrt(cnt)
    g2 = g2_ref[...]
    x1 = g2 * sq[:, None]
    x2 = (s2_ref[0] + s2_ref[1] + g2) * dis[:, None]
    dot = functools.partial(jnp.dot, preferred_element_type=jnp.float32)
    t0 = jnp.maximum(dot(x_ref[...], w0_ref[...]) + b0_ref[...], 0.0)
    t1 = jnp.maximum(dot(x1, w1_ref[...]) + b1_ref[...], 0.0)
    t2 = jnp.maximum(dot(x2, w2_ref[...]) + b2_ref[...], 0.0)
    wf = wf_ref[...]
    out = dot(t0, wf[:64]) + dot(t1, wf[64:128]) + dot(t2, wf[128:]) \
        + bf_ref[...]
    o_ref[...] = out


def _spec_d():
    return pl.BlockSpec((2, _BN), lambda i: (0, i))


def _spec_rows():
    return pl.BlockSpec((_BN, D), lambda i: (i, 0))


def _spec_pair():
    return pl.BlockSpec((2, _BN, D), lambda i: (0, i, 0))


def _spec_full(shape):
    return pl.BlockSpec(shape, lambda i: tuple(0 for _ in shape))


def kernel(x, edge_index, W0, b0, W1, b1, W2, b2, Wf, bf):
    n, d = x.shape
    f32 = jnp.float32

    x_pad = jnp.pad(x, ((0, NPAD - n), (0, 0)))
    e = edge_index.shape[1]
    i32 = jnp.int32
    row_rs = jnp.concatenate(
        [edge_index[0], jnp.zeros((EPAD - e,), i32)]).reshape(32, NCH2, CH2)
    col_rs = jnp.concatenate(
        [edge_index[1], jnp.full((EPAD - e,), NPAD - 1, i32)]
    ).reshape(32, NCH2, CH2)
    col_dg = edge_index[1].reshape(32, NCH, CH)
    z1 = jnp.zeros((NPAD,), f32)
    z2 = jnp.zeros((NPAD, D), f32)
    ones = jnp.ones((CH,), f32)

    deg2 = _deg_kernel(col_dg, ones, z1)

    g1 = pl.pallas_call(
        _tc1_body,
        grid=(_GRID,),
        in_specs=[_spec_d(), _spec_rows()],
        out_specs=_spec_rows(),
        out_shape=jax.ShapeDtypeStruct((NPAD, D), f32),
    )(deg2, x_pad)

    s1p = _hop_kernel(row_rs, col_rs, g1, z2)

    g2 = pl.pallas_call(
        _tc2_body,
        grid=(_GRID,),
        in_specs=[_spec_d(), _spec_pair(), _spec_rows()],
        out_specs=_spec_rows(),
        out_shape=jax.ShapeDtypeStruct((NPAD, D), f32),
    )(deg2, s1p, g1)

    s2p = _hop_kernel(row_rs, col_rs, g2, z2)

    out = pl.pallas_call(
        _tc3_body,
        grid=(_GRID,),
        in_specs=[
            _spec_d(), _spec_rows(), _spec_rows(), _spec_pair(),
            _spec_full((D, 64)), _spec_full((1, 64)),
            _spec_full((D, 64)), _spec_full((1, 64)),
            _spec_full((D, 64)), _spec_full((1, 64)),
            _spec_full((192, D)), _spec_full((1, D)),
        ],
        out_specs=_spec_rows(),
        out_shape=jax.ShapeDtypeStruct((NPAD, D), f32),
    )(deg2, x_pad, g2, s2p, W0, b0.reshape(1, 64), W1, b1.reshape(1, 64),
      W2, b2.reshape(1, 64), Wf, bf.reshape(1, D))

    return out[:n]


# speedup vs baseline: 2.5334x; 2.5334x over previous
import functools

import jax
import jax.numpy as jnp
from jax import lax
from jax.experimental import pallas as pl
from jax.experimental.pallas import tpu as pltpu
from jax.experimental.pallas import tpu_sc as plsc

NPAD = 10240
D = 128
CH = 80
NCH = 125
CH2 = 128
NCH2 = 80
EPAD = 32 * NCH2 * CH2
ROWS_PER_TILE = 640

_mesh = plsc.VectorSubcoreMesh(core_axis_name="c", subcore_axis_name="s")


@functools.partial(
    pl.kernel,
    out_type=jax.ShapeDtypeStruct((2, NPAD), jnp.float32),
    mesh=_mesh,
    scratch_types=[
        pltpu.VMEM((NCH, CH), jnp.int32),
        pltpu.VMEM((CH,), jnp.float32),
        pltpu.VMEM_SHARED((NPAD,), jnp.float32),
    ],
)
def _deg_kernel(col_ref, ones_ref, z1_ref, out_ref, col_v, ones_v, deg_sh):
    cid = lax.axis_index("c")
    sid = lax.axis_index("s")
    pltpu.sync_copy(ones_ref, ones_v)
    pltpu.sync_copy(col_ref.at[cid * 16 + sid], col_v)
    sl = pl.ds(sid * ROWS_PER_TILE, ROWS_PER_TILE)
    pltpu.sync_copy(z1_ref.at[sl], deg_sh.at[sl])
    plsc.subcore_barrier()

    def body(i, carry):
        pltpu.sync_copy(ones_v, deg_sh.at[col_v.at[i]], add=True)
        return carry

    lax.fori_loop(0, NCH, body, 0)
    plsc.subcore_barrier()
    pltpu.sync_copy(deg_sh.at[sl], out_ref.at[cid, sl])


@functools.partial(
    pl.kernel,
    out_type=jax.ShapeDtypeStruct((2, NPAD, D), jnp.float32),
    mesh=_mesh,
    scratch_types=[
        pltpu.VMEM((CH2,), jnp.int32),
        pltpu.VMEM((CH2,), jnp.int32),
        pltpu.VMEM((CH2,), jnp.int32),
        pltpu.VMEM((CH2,), jnp.int32),
        pltpu.VMEM((CH2, D), jnp.float32),
        pltpu.VMEM((CH2, D), jnp.float32),
        pltpu.VMEM_SHARED((NPAD, D), jnp.float32),
        pltpu.SemaphoreType.DMA,
        pltpu.SemaphoreType.DMA,
        pltpu.SemaphoreType.DMA,
        pltpu.SemaphoreType.DMA,
        pltpu.SemaphoreType.DMA,
        pltpu.SemaphoreType.DMA,
        pltpu.SemaphoreType.DMA,
        pltpu.SemaphoreType.DMA,
    ],
)
def _hop_kernel(row_ref, col_ref, g_ref, z2_ref, out_ref,
                ridx0, ridx1, cidx0, cidx1, rows0, rows1, acc_sh,
                gi0r, gi1r, gi0c, gi1c, gg0, gg1, gs0, gs1):
    cid = lax.axis_index("c")
    sid = lax.axis_index("s")
    wid = cid * 16 + sid
    sl = pl.ds(sid * ROWS_PER_TILE, ROWS_PER_TILE)
    pltpu.sync_copy(z2_ref.at[sl], acc_sh.at[sl])
    plsc.subcore_barrier()

    pltpu.async_copy(row_ref.at[wid, 0], ridx0, gi0r)
    pltpu.async_copy(col_ref.at[wid, 0], cidx0, gi0c)
    pltpu.async_copy(row_ref.at[wid, 1], ridx1, gi1r)
    pltpu.async_copy(col_ref.at[wid, 1], cidx1, gi1c)
    pltpu.make_async_copy(row_ref.at[wid, 0], ridx0, gi0r).wait()
    pltpu.async_copy(g_ref.at[ridx0], rows0, gg0)
    pltpu.make_async_copy(row_ref.at[wid, 1], ridx1, gi1r).wait()
    pltpu.async_copy(g_ref.at[ridx1], rows1, gg1)

    def body(i, carry):
        c0 = 2 * i
        c1 = c0 + 1
        pltpu.make_async_copy(g_ref.at[ridx0], rows0, gg0).wait()
        pltpu.make_async_copy(col_ref.at[wid, c0], cidx0, gi0c).wait()
        pltpu.async_copy(rows0, acc_sh.at[cidx0], gs0, add=True)
        pltpu.async_copy(row_ref.at[wid, c0 + 2], ridx0, gi0r)
        pltpu.make_async_copy(g_ref.at[ridx1], rows1, gg1).wait()
        pltpu.make_async_copy(col_ref.at[wid, c1], cidx1, gi1c).wait()
        pltpu.async_copy(rows1, acc_sh.at[cidx1], gs1, add=True)
        pltpu.async_copy(row_ref.at[wid, c1 + 2], ridx1, gi1r)
        pltpu.make_async_copy(rows0, acc_sh.at[cidx0], gs0).wait()
        pltpu.async_copy(col_ref.at[wid, c0 + 2], cidx0, gi0c)
        pltpu.make_async_copy(row_ref.at[wid, c0 + 2], ridx0, gi0r).wait()
        pltpu.async_copy(g_ref.at[ridx0], rows0, gg0)
        pltpu.make_async_copy(rows1, acc_sh.at[cidx1], gs1).wait()
        pltpu.async_copy(col_ref.at[wid, c1 + 2], cidx1, gi1c)
        pltpu.make_async_copy(row_ref.at[wid, c1 + 2], ridx1, gi1r).wait()
        pltpu.async_copy(g_ref.at[ridx1], rows1, gg1)
        return carry

    lax.fori_loop(0, NCH2 // 2 - 1, body, 0)
    cl0 = NCH2 - 2
    cl1 = NCH2 - 1
    pltpu.make_async_copy(g_ref.at[ridx0], rows0, gg0).wait()
    pltpu.make_async_copy(col_ref.at[wid, cl0], cidx0, gi0c).wait()
    pltpu.async_copy(rows0, acc_sh.at[cidx0], gs0, add=True)
    pltpu.make_async_copy(g_ref.at[ridx1], rows1, gg1).wait()
    pltpu.make_async_copy(col_ref.at[wid, cl1], cidx1, gi1c).wait()
    pltpu.async_copy(rows1, acc_sh.at[cidx1], gs1, add=True)
    pltpu.make_async_copy(rows0, acc_sh.at[cidx0], gs0).wait()
    pltpu.make_async_copy(rows1, acc_sh.at[cidx1], gs1).wait()
    plsc.subcore_barrier()
    pltpu.sync_copy(acc_sh.at[sl], out_ref.at[cid, sl])


_BN = 512
_GRID = NPAD // _BN


def _tc1_body(d_ref, x_ref, o_ref):
    cnt = d_ref[0, :] + d_ref[1, :] + 1.0
    dis = lax.rsqrt(cnt)
    o_ref[...] = x_ref[...] * dis[:, None]


def _tc2_body(d_ref, s_ref, g_ref, o_ref):
    cnt = d_ref[0, :] + d_ref[1, :] + 1.0
    inv = 1.0 / cnt
    o_ref[...] = (s_ref[0] + s_ref[1] + g_ref[...]) * inv[:, None]


def _tc3_body(d_ref, x_ref, g2_ref, s2_ref, w0_ref, b0_ref, w1_ref, b1_ref,
              w2_ref, b2_ref, wf_ref, bf_ref, o_ref):
    cnt = d_ref[0, :] + d_ref[1, :] + 1.0
    dis = lax.rsqrt(cnt)
    sq = jnp.sqrt(cnt)
    g2 = g2_ref[...]
    x1 = g2 * sq[:, None]
    x2 = (s2_ref[0] + s2_ref[1] + g2) * dis[:, None]
    dot = functools.partial(jnp.dot, preferred_element_type=jnp.float32)
    t0 = jnp.maximum(dot(x_ref[...], w0_ref[...]) + b0_ref[...], 0.0)
    t1 = jnp.maximum(dot(x1, w1_ref[...]) + b1_ref[...], 0.0)
    t2 = jnp.maximum(dot(x2, w2_ref[...]) + b2_ref[...], 0.0)
    wf = wf_ref[...]
    out = dot(t0, wf[:64]) + dot(t1, wf[64:128]) + dot(t2, wf[128:]) \
        + bf_ref[...]
    o_ref[...] = out


def _spec_d():
    return pl.BlockSpec((2, _BN), lambda i: (0, i))


def _spec_rows():
    return pl.BlockSpec((_BN, D), lambda i: (i, 0))


def _spec_pair():
    return pl.BlockSpec((2, _BN, D), lambda i: (0, i, 0))


def _spec_full(shape):
    return pl.BlockSpec(shape, lambda i: tuple(0 for _ in shape))


def kernel(x, edge_index, W0, b0, W1, b1, W2, b2, Wf, bf):
    n, d = x.shape
    f32 = jnp.float32

    x_pad = jnp.pad(x, ((0, NPAD - n), (0, 0)))
    e = edge_index.shape[1]
    i32 = jnp.int32
    pidx = jnp.arange(EPAD - e, dtype=i32)
    row_rs = jnp.concatenate(
        [edge_index[0], pidx % n]).reshape(32, NCH2, CH2)
    col_rs = jnp.concatenate(
        [edge_index[1], n + pidx % (NPAD - n)]).reshape(32, NCH2, CH2)
    col_dg = edge_index[1].reshape(32, NCH, CH)
    z1 = jnp.zeros((NPAD,), f32)
    z2 = jnp.zeros((NPAD, D), f32)
    ones = jnp.ones((CH,), f32)

    deg2 = _deg_kernel(col_dg, ones, z1)

    g1 = pl.pallas_call(
        _tc1_body,
        grid=(_GRID,),
        in_specs=[_spec_d(), _spec_rows()],
        out_specs=_spec_rows(),
        out_shape=jax.ShapeDtypeStruct((NPAD, D), f32),
    )(deg2, x_pad)

    s1p = _hop_kernel(row_rs, col_rs, g1, z2)

    g2 = pl.pallas_call(
        _tc2_body,
        grid=(_GRID,),
        in_specs=[_spec_d(), _spec_pair(), _spec_rows()],
        out_specs=_spec_rows(),
        out_shape=jax.ShapeDtypeStruct((NPAD, D), f32),
    )(deg2, s1p, g1)

    s2p = _hop_kernel(row_rs, col_rs, g2, z2)

    out = pl.pallas_call(
        _tc3_body,
        grid=(_GRID,),
        in_specs=[
            _spec_d(), _spec_rows(), _spec_rows(), _spec_pair(),
            _spec_full((D, 64)), _spec_full((1, 64)),
            _spec_full((D, 64)), _spec_full((1, 64)),
            _spec_full((D, 64)), _spec_full((1, 64)),
            _spec_full((192, D)), _spec_full((1, D)),
        ],
        out_specs=_spec_rows(),
        out_shape=jax.ShapeDtypeStruct((NPAD, D), f32),
    )(deg2, x_pad, g2, s2p, W0, b0.reshape(1, 64), W1, b1.reshape(1, 64),
      W2, b2.reshape(1, 64), Wf, bf.reshape(1, D))

    return out[:n]


# speedup vs baseline: 2.8136x; 1.1106x over previous
import functools

import jax
import jax.numpy as jnp
from jax import lax
from jax.experimental import pallas as pl
from jax.experimental.pallas import tpu as pltpu
from jax.experimental.pallas import tpu_sc as plsc

NPAD = 10112
D = 128
CH = 80
NCH = 125
NBUF = 3
CH2 = 128
NCH2 = 81
EPAD = 32 * NCH2 * CH2
ROWS_PER_TILE = 632

_mesh = plsc.VectorSubcoreMesh(core_axis_name="c", subcore_axis_name="s")


@functools.partial(
    pl.kernel,
    out_type=jax.ShapeDtypeStruct((2 * NPAD,), jnp.float32),
    mesh=_mesh,
    scratch_types=[
        pltpu.VMEM((NCH, CH), jnp.int32),
        pltpu.VMEM((CH,), jnp.float32),
        pltpu.VMEM_SHARED((NPAD,), jnp.float32),
    ],
)
def _deg_kernel(col_ref, ones_ref, z1_ref, out_ref, col_v, ones_v, deg_sh):
    cid = lax.axis_index("c")
    sid = lax.axis_index("s")
    pltpu.sync_copy(ones_ref, ones_v)
    pltpu.sync_copy(col_ref.at[cid * 16 + sid], col_v)
    sl_a = pl.ds(sid * 640, 640)
    sl_b = pl.ds(15 * 640, 512)

    @pl.when(sid < 15)
    def _():
        pltpu.sync_copy(z1_ref.at[sl_a], deg_sh.at[sl_a])

    @pl.when(sid == 15)
    def _():
        pltpu.sync_copy(z1_ref.at[sl_b], deg_sh.at[sl_b])

    plsc.subcore_barrier()

    def body(i, carry):
        pltpu.sync_copy(ones_v, deg_sh.at[col_v.at[i]], add=True)
        return carry

    lax.fori_loop(0, NCH, body, 0)
    plsc.subcore_barrier()

    @pl.when(sid < 15)
    def _():
        pltpu.sync_copy(deg_sh.at[sl_a],
                        out_ref.at[pl.ds(cid * NPAD + sid * 640, 640)])

    @pl.when(sid == 15)
    def _():
        pltpu.sync_copy(deg_sh.at[sl_b],
                        out_ref.at[pl.ds(cid * NPAD + 15 * 640, 512)])


@functools.partial(
    pl.kernel,
    out_type=jax.ShapeDtypeStruct((2, NPAD, D), jnp.float32),
    mesh=_mesh,
    scratch_types=(
        [pltpu.VMEM((CH2,), jnp.int32) for _ in range(2 * NBUF)]
        + [pltpu.VMEM((CH2, D), jnp.float32) for _ in range(NBUF)]
        + [pltpu.VMEM_SHARED((NPAD, D), jnp.float32)]
        + [pltpu.SemaphoreType.DMA for _ in range(4 * NBUF)]
    ),
)
def _hop_kernel(row_ref, col_ref, g_ref, z2_ref, out_ref, *scr):
    ridx = scr[0:NBUF]
    cidx = scr[NBUF:2 * NBUF]
    rows = scr[2 * NBUF:3 * NBUF]
    acc_sh = scr[3 * NBUF]
    gir = scr[3 * NBUF + 1:3 * NBUF + 1 + NBUF]
    gic = scr[3 * NBUF + 1 + NBUF:3 * NBUF + 1 + 2 * NBUF]
    gg = scr[3 * NBUF + 1 + 2 * NBUF:3 * NBUF + 1 + 3 * NBUF]
    gs = scr[3 * NBUF + 1 + 3 * NBUF:3 * NBUF + 1 + 4 * NBUF]
    cid = lax.axis_index("c")
    sid = lax.axis_index("s")
    wid = cid * 16 + sid
    sl = pl.ds(sid * ROWS_PER_TILE, ROWS_PER_TILE)
    pltpu.sync_copy(z2_ref.at[sl], acc_sh.at[sl])
    plsc.subcore_barrier()

    for p in range(NBUF):
        pltpu.async_copy(row_ref.at[p, wid, 0], ridx[p], gir[p])
        pltpu.async_copy(col_ref.at[p, wid, 0], cidx[p], gic[p])
    for p in range(NBUF):
        pltpu.make_async_copy(row_ref.at[p, wid, 0], ridx[p], gir[p]).wait()
        pltpu.async_copy(g_ref.at[ridx[p]], rows[p], gg[p])

    def body(i, carry):
        c = NBUF * i
        for p in range(NBUF):
            pltpu.make_async_copy(g_ref.at[ridx[p]], rows[p], gg[p]).wait()
            pltpu.make_async_copy(
                col_ref.at[c + p, wid, 0], cidx[p], gic[p]).wait()
            pltpu.async_copy(rows[p], acc_sh.at[cidx[p]], gs[p], add=True)
            pltpu.async_copy(
                row_ref.at[c + p + NBUF, wid, 0], ridx[p], gir[p])
        for p in range(NBUF):
            pltpu.make_async_copy(rows[p], acc_sh.at[cidx[p]], gs[p]).wait()
            pltpu.async_copy(
                col_ref.at[c + p + NBUF, wid, 0], cidx[p], gic[p])
            pltpu.make_async_copy(
                row_ref.at[c + p + NBUF, wid, 0], ridx[p], gir[p]).wait()
            pltpu.async_copy(g_ref.at[ridx[p]], rows[p], gg[p])
        return carry

    lax.fori_loop(0, NCH2 // NBUF - 1, body, 0)
    ce = NCH2 - NBUF
    for p in range(NBUF):
        pltpu.make_async_copy(g_ref.at[ridx[p]], rows[p], gg[p]).wait()
        pltpu.make_async_copy(
            col_ref.at[ce + p, wid, 0], cidx[p], gic[p]).wait()
        pltpu.async_copy(rows[p], acc_sh.at[cidx[p]], gs[p], add=True)
    for p in range(NBUF):
        pltpu.make_async_copy(rows[p], acc_sh.at[cidx[p]], gs[p]).wait()
    plsc.subcore_barrier()
    pltpu.sync_copy(acc_sh.at[sl], out_ref.at[cid, sl])


_BN = 632
_GRID = NPAD // _BN
NREAL = 10000


def _tc1_body(d_ref, x_ref, o_ref):
    cnt = d_ref[:, 0] + d_ref[:, 1] + 1.0
    dis = lax.rsqrt(cnt)
    o_ref[...] = x_ref[...] * dis[:, None]


def _tc2_body(d_ref, s_ref, g_ref, o_ref):
    cnt = d_ref[:, 0] + d_ref[:, 1] + 1.0
    inv = 1.0 / cnt
    r = pl.program_id(0) * _BN + lax.broadcasted_iota(jnp.int32, (_BN, 1), 0)
    o_ref[...] = jnp.where(
        r < NREAL, (s_ref[0] + s_ref[1] + g_ref[...]) * inv[:, None], 0.0)


def _tc3_body(d_ref, x_ref, g2_ref, s2_ref, w0_ref, b0_ref, w1_ref, b1_ref,
              w2_ref, b2_ref, wf_ref, bf_ref, o_ref):
    cnt = d_ref[:, 0] + d_ref[:, 1] + 1.0
    dis = lax.rsqrt(cnt)
    sq = jnp.sqrt(cnt)
    g2 = g2_ref[...]
    x1 = g2 * sq[:, None]
    x2 = (s2_ref[0] + s2_ref[1] + g2) * dis[:, None]
    dot = functools.partial(jnp.dot, preferred_element_type=jnp.float32)
    t0 = jnp.maximum(dot(x_ref[...], w0_ref[...]) + b0_ref[...], 0.0)
    t1 = jnp.maximum(dot(x1, w1_ref[...]) + b1_ref[...], 0.0)
    t2 = jnp.maximum(dot(x2, w2_ref[...]) + b2_ref[...], 0.0)
    wf = wf_ref[...]
    out = dot(t0, wf[:64]) + dot(t1, wf[64:128]) + dot(t2, wf[128:]) \
        + bf_ref[...]
    o_ref[...] = out


def _spec_d():
    return pl.BlockSpec((_BN, 2), lambda i: (i, 0))


def _spec_rows():
    return pl.BlockSpec((_BN, D), lambda i: (i, 0))


def _spec_pair():
    return pl.BlockSpec((2, _BN, D), lambda i: (0, i, 0))


def _spec_full(shape):
    return pl.BlockSpec(shape, lambda i: tuple(0 for _ in shape))


def kernel(x, edge_index, W0, b0, W1, b1, W2, b2, Wf, bf):
    n, d = x.shape
    f32 = jnp.float32

    x_pad = jnp.pad(x, ((0, NPAD - n), (0, 0)))
    e = edge_index.shape[1]
    i32 = jnp.int32
    pidx = jnp.arange(EPAD - e, dtype=i32)
    row_rs = jnp.concatenate(
        [edge_index[0], n + pidx % (NPAD - n)]
    ).reshape(32, NCH2, 1, CH2).transpose(1, 0, 2, 3)
    col_rs = jnp.concatenate(
        [edge_index[1], pidx % NPAD]
    ).reshape(32, NCH2, 1, CH2).transpose(1, 0, 2, 3)
    col_dg = edge_index[1].reshape(32, NCH, CH)
    z1 = jnp.zeros((NPAD,), f32)
    z2 = jnp.zeros((NPAD, D), f32)
    ones = jnp.ones((CH,), f32)

    deg2 = _deg_kernel(col_dg, ones, z1).reshape(2, NPAD).T

    g1 = pl.pallas_call(
        _tc1_body,
        grid=(_GRID,),
        in_specs=[_spec_d(), _spec_rows()],
        out_specs=_spec_rows(),
        out_shape=jax.ShapeDtypeStruct((NPAD, D), f32),
    )(deg2, x_pad)

    s1p = _hop_kernel(row_rs, col_rs, g1, z2)

    g2 = pl.pallas_call(
        _tc2_body,
        grid=(_GRID,),
        in_specs=[_spec_d(), _spec_pair(), _spec_rows()],
        out_specs=_spec_rows(),
        out_shape=jax.ShapeDtypeStruct((NPAD, D), f32),
    )(deg2, s1p, g1)

    s2p = _hop_kernel(row_rs, col_rs, g2, z2)

    out = pl.pallas_call(
        _tc3_body,
        grid=(_GRID,),
        in_specs=[
            _spec_d(), _spec_rows(), _spec_rows(), _spec_pair(),
            _spec_full((D, 64)), _spec_full((1, 64)),
            _spec_full((D, 64)), _spec_full((1, 64)),
            _spec_full((D, 64)), _spec_full((1, 64)),
            _spec_full((192, D)), _spec_full((1, D)),
        ],
        out_specs=_spec_rows(),
        out_shape=jax.ShapeDtypeStruct((NPAD, D), f32),
    )(deg2, x_pad, g2, s2p, W0, b0.reshape(1, 64), W1, b1.reshape(1, 64),
      W2, b2.reshape(1, 64), Wf, bf.reshape(1, D))

    return out[:n]


# speedup vs baseline: 2.9174x; 1.0369x over previous
import functools

import jax
import jax.numpy as jnp
from jax import lax
from jax.experimental import pallas as pl
from jax.experimental.pallas import tpu as pltpu
from jax.experimental.pallas import tpu_sc as plsc

NPAD = 10112
D = 128
CH = 128
NCH = 80
NBUF = 3
CH2 = 128
NCH2 = 81
EPAD = 32 * NCH2 * CH2
ROWS_PER_TILE = 632

_mesh = plsc.VectorSubcoreMesh(core_axis_name="c", subcore_axis_name="s")


@functools.partial(
    pl.kernel,
    out_type=jax.ShapeDtypeStruct((2 * NPAD,), jnp.float32),
    mesh=_mesh,
    scratch_types=[
        pltpu.VMEM((NCH, CH), jnp.int32),
        pltpu.VMEM((CH,), jnp.float32),
        pltpu.VMEM_SHARED((NPAD,), jnp.float32),
        pltpu.SemaphoreType.DMA,
        pltpu.SemaphoreType.DMA,
    ],
)
def _deg_kernel(col_ref, ones_ref, z1_ref, out_ref, col_v, ones_v, deg_sh,
                d0, d1):
    cid = lax.axis_index("c")
    sid = lax.axis_index("s")
    pltpu.sync_copy(ones_ref, ones_v)
    pltpu.sync_copy(col_ref.at[cid * 16 + sid], col_v)
    sl_a = pl.ds(sid * 640, 640)
    sl_b = pl.ds(15 * 640, 512)

    @pl.when(sid < 15)
    def _():
        pltpu.sync_copy(z1_ref.at[sl_a], deg_sh.at[sl_a])

    @pl.when(sid == 15)
    def _():
        pltpu.sync_copy(z1_ref.at[sl_b], deg_sh.at[sl_b])

    plsc.subcore_barrier()

    def body(i, carry):
        pltpu.async_copy(ones_v, deg_sh.at[col_v.at[2 * i]], d0, add=True)
        pltpu.async_copy(ones_v, deg_sh.at[col_v.at[2 * i + 1]], d1, add=True)
        pltpu.make_async_copy(ones_v, deg_sh.at[col_v.at[2 * i]], d0).wait()
        pltpu.make_async_copy(
            ones_v, deg_sh.at[col_v.at[2 * i + 1]], d1).wait()
        return carry

    lax.fori_loop(0, NCH // 2, body, 0)
    plsc.subcore_barrier()

    @pl.when(sid < 15)
    def _():
        pltpu.sync_copy(deg_sh.at[sl_a],
                        out_ref.at[pl.ds(cid * NPAD + sid * 640, 640)])

    @pl.when(sid == 15)
    def _():
        pltpu.sync_copy(deg_sh.at[sl_b],
                        out_ref.at[pl.ds(cid * NPAD + 15 * 640, 512)])


@functools.partial(
    pl.kernel,
    out_type=jax.ShapeDtypeStruct((2, NPAD, D), jnp.float32),
    mesh=_mesh,
    scratch_types=(
        [pltpu.VMEM((CH2,), jnp.int32) for _ in range(2 * NBUF)]
        + [pltpu.VMEM((CH2, D), jnp.float32) for _ in range(NBUF)]
        + [pltpu.VMEM_SHARED((NPAD, D), jnp.float32)]
        + [pltpu.SemaphoreType.DMA for _ in range(4 * NBUF)]
    ),
)
def _hop_kernel(row_ref, col_ref, g_ref, z2_ref, out_ref, *scr):
    ridx = scr[0:NBUF]
    cidx = scr[NBUF:2 * NBUF]
    rows = scr[2 * NBUF:3 * NBUF]
    acc_sh = scr[3 * NBUF]
    gir = scr[3 * NBUF + 1:3 * NBUF + 1 + NBUF]
    gic = scr[3 * NBUF + 1 + NBUF:3 * NBUF + 1 + 2 * NBUF]
    gg = scr[3 * NBUF + 1 + 2 * NBUF:3 * NBUF + 1 + 3 * NBUF]
    gs = scr[3 * NBUF + 1 + 3 * NBUF:3 * NBUF + 1 + 4 * NBUF]
    cid = lax.axis_index("c")
    sid = lax.axis_index("s")
    wid = cid * 16 + sid
    sl = pl.ds(sid * ROWS_PER_TILE, ROWS_PER_TILE)
    pltpu.sync_copy(z2_ref.at[sl], acc_sh.at[sl])
    plsc.subcore_barrier()

    for p in range(NBUF):
        pltpu.async_copy(row_ref.at[p, wid, 0], ridx[p], gir[p])
        pltpu.async_copy(col_ref.at[p, wid, 0], cidx[p], gic[p])
    for p in range(NBUF):
        pltpu.make_async_copy(row_ref.at[p, wid, 0], ridx[p], gir[p]).wait()
        pltpu.async_copy(g_ref.at[ridx[p]], rows[p], gg[p])

    def body(i, carry):
        c = NBUF * i
        for p in range(NBUF):
            pltpu.make_async_copy(g_ref.at[ridx[p]], rows[p], gg[p]).wait()
            pltpu.make_async_copy(
                col_ref.at[c + p, wid, 0], cidx[p], gic[p]).wait()
            pltpu.async_copy(rows[p], acc_sh.at[cidx[p]], gs[p], add=True)
            pltpu.async_copy(
                row_ref.at[c + p + NBUF, wid, 0], ridx[p], gir[p])
        for p in range(NBUF):
            pltpu.make_async_copy(rows[p], acc_sh.at[cidx[p]], gs[p]).wait()
            pltpu.async_copy(
                col_ref.at[c + p + NBUF, wid, 0], cidx[p], gic[p])
            pltpu.make_async_copy(
                row_ref.at[c + p + NBUF, wid, 0], ridx[p], gir[p]).wait()
            pltpu.async_copy(g_ref.at[ridx[p]], rows[p], gg[p])
        return carry

    lax.fori_loop(0, NCH2 // NBUF - 1, body, 0)
    ce = NCH2 - NBUF
    for p in range(NBUF):
        pltpu.make_async_copy(g_ref.at[ridx[p]], rows[p], gg[p]).wait()
        pltpu.make_async_copy(
            col_ref.at[ce + p, wid, 0], cidx[p], gic[p]).wait()
        pltpu.async_copy(rows[p], acc_sh.at[cidx[p]], gs[p], add=True)
    for p in range(NBUF):
        pltpu.make_async_copy(rows[p], acc_sh.at[cidx[p]], gs[p]).wait()
    plsc.subcore_barrier()
    pltpu.sync_copy(acc_sh.at[sl], out_ref.at[cid, sl])


_BN = 632
_GRID = NPAD // _BN
NREAL = 10000


def _tc1_body(d_ref, x_ref, o_ref):
    cnt = d_ref[:, 0] + d_ref[:, 1] + 1.0
    dis = lax.rsqrt(cnt)
    r = pl.program_id(0) * _BN + lax.broadcasted_iota(jnp.int32, (_BN, 1), 0)
    o_ref[...] = jnp.where(r < NREAL, x_ref[...] * dis[:, None], 0.0)


def _tc2_body(d_ref, s_ref, g_ref, o_ref):
    cnt = d_ref[:, 0] + d_ref[:, 1] + 1.0
    inv = 1.0 / cnt
    r = pl.program_id(0) * _BN + lax.broadcasted_iota(jnp.int32, (_BN, 1), 0)
    o_ref[...] = jnp.where(
        r < NREAL, (s_ref[0] + s_ref[1] + g_ref[...]) * inv[:, None], 0.0)


def _tc3_body(d_ref, x_ref, g2_ref, s2_ref, w0_ref, b0_ref, w1_ref, b1_ref,
              w2_ref, b2_ref, wf_ref, bf_ref, o_ref):
    cnt = d_ref[:, 0] + d_ref[:, 1] + 1.0
    dis = lax.rsqrt(cnt)
    sq = jnp.sqrt(cnt)
    g2 = g2_ref[...]
    x1 = g2 * sq[:, None]
    x2 = (s2_ref[0] + s2_ref[1] + g2) * dis[:, None]
    dot = functools.partial(jnp.dot, preferred_element_type=jnp.float32)
    t0 = jnp.maximum(dot(x_ref[...], w0_ref[...]) + b0_ref[...], 0.0)
    t1 = jnp.maximum(dot(x1, w1_ref[...]) + b1_ref[...], 0.0)
    t2 = jnp.maximum(dot(x2, w2_ref[...]) + b2_ref[...], 0.0)
    wf = wf_ref[...]
    out = dot(t0, wf[:64]) + dot(t1, wf[64:128]) + dot(t2, wf[128:]) \
        + bf_ref[...]
    o_ref[...] = out


def _spec_d():
    return pl.BlockSpec((_BN, 2), lambda i: (i, 0))


def _spec_rows():
    return pl.BlockSpec((_BN, D), lambda i: (i, 0))


def _spec_pair():
    return pl.BlockSpec((2, _BN, D), lambda i: (0, i, 0))


def _spec_full(shape):
    return pl.BlockSpec(shape, lambda i: tuple(0 for _ in shape))


def kernel(x, edge_index, W0, b0, W1, b1, W2, b2, Wf, bf):
    n, d = x.shape
    f32 = jnp.float32

    e = edge_index.shape[1]
    i32 = jnp.int32
    pidx = jnp.arange(EPAD - e, dtype=i32)
    row_rs = jnp.concatenate(
        [edge_index[0], n + pidx % (NPAD - n)]).reshape(NCH2, 32, 1, CH2)
    col_rs = jnp.concatenate(
        [edge_index[1], pidx % NPAD]).reshape(NCH2, 32, 1, CH2)
    pidx2 = jnp.arange(32 * NCH * CH - e, dtype=i32)
    col_dg = jnp.concatenate(
        [edge_index[1], n + pidx2 % (NPAD - n)]).reshape(32, NCH, CH)
    z1 = jnp.zeros((NPAD,), f32)
    z2 = jnp.zeros((NPAD, D), f32)
    ones = jnp.ones((CH,), f32)

    deg2 = _deg_kernel(col_dg, ones, z1).reshape(2, NPAD).T

    g1 = pl.pallas_call(
        _tc1_body,
        grid=(_GRID,),
        in_specs=[_spec_d(), _spec_rows()],
        out_specs=_spec_rows(),
        out_shape=jax.ShapeDtypeStruct((NPAD, D), f32),
    )(deg2, x)

    s1p = _hop_kernel(row_rs, col_rs, g1, z2)

    g2 = pl.pallas_call(
        _tc2_body,
        grid=(_GRID,),
        in_specs=[_spec_d(), _spec_pair(), _spec_rows()],
        out_specs=_spec_rows(),
        out_shape=jax.ShapeDtypeStruct((NPAD, D), f32),
    )(deg2, s1p, g1)

    s2p = _hop_kernel(row_rs, col_rs, g2, z2)

    out = pl.pallas_call(
        _tc3_body,
        grid=(_GRID,),
        in_specs=[
            _spec_d(), _spec_rows(), _spec_rows(), _spec_pair(),
            _spec_full((D, 64)), _spec_full((1, 64)),
            _spec_full((D, 64)), _spec_full((1, 64)),
            _spec_full((D, 64)), _spec_full((1, 64)),
            _spec_full((192, D)), _spec_full((1, D)),
        ],
        out_specs=_spec_rows(),
        out_shape=jax.ShapeDtypeStruct((n, D), f32),
    )(deg2, x, g2, s2p, W0, b0.reshape(1, 64), W1, b1.reshape(1, 64),
      W2, b2.reshape(1, 64), Wf, bf.reshape(1, D))

    return out


# speedup vs baseline: 2.9717x; 1.0186x over previous
import functools

import jax
import jax.numpy as jnp
from jax import lax
from jax.experimental import pallas as pl
from jax.experimental.pallas import tpu as pltpu
from jax.experimental.pallas import tpu_sc as plsc

NPAD = 10112
D = 128
CH = 128
NCH = 80
NBUF = 3
CH2 = 128
NCH2 = 81
EPAD = 32 * NCH2 * CH2
ROWS_PER_TILE = 632

_mesh = plsc.VectorSubcoreMesh(core_axis_name="c", subcore_axis_name="s")


@functools.partial(
    pl.kernel,
    out_type=jax.ShapeDtypeStruct((2 * NPAD,), jnp.float32),
    mesh=_mesh,
    scratch_types=[
        pltpu.VMEM((NCH, CH), jnp.int32),
        pltpu.VMEM((CH,), jnp.float32),
        pltpu.VMEM_SHARED((NPAD,), jnp.float32),
        pltpu.SemaphoreType.DMA,
        pltpu.SemaphoreType.DMA,
    ],
)
def _deg_kernel(col_ref, ones_ref, z1_ref, out_ref, col_v, ones_v, deg_sh,
                d0, d1):
    cid = lax.axis_index("c")
    sid = lax.axis_index("s")
    pltpu.sync_copy(ones_ref, ones_v)
    pltpu.sync_copy(col_ref.at[cid * 16 + sid], col_v)
    sl_a = pl.ds(sid * 640, 640)
    sl_b = pl.ds(15 * 640, 512)

    @pl.when(sid < 15)
    def _():
        pltpu.sync_copy(z1_ref.at[sl_a], deg_sh.at[sl_a])

    @pl.when(sid == 15)
    def _():
        pltpu.sync_copy(z1_ref.at[sl_b], deg_sh.at[sl_b])

    plsc.subcore_barrier()

    def body(i, carry):
        pltpu.async_copy(ones_v, deg_sh.at[col_v.at[2 * i]], d0, add=True)
        pltpu.async_copy(ones_v, deg_sh.at[col_v.at[2 * i + 1]], d1, add=True)
        pltpu.make_async_copy(ones_v, deg_sh.at[col_v.at[2 * i]], d0).wait()
        pltpu.make_async_copy(
            ones_v, deg_sh.at[col_v.at[2 * i + 1]], d1).wait()
        return carry

    lax.fori_loop(0, NCH // 2, body, 0)
    plsc.subcore_barrier()

    @pl.when(sid < 15)
    def _():
        pltpu.sync_copy(deg_sh.at[sl_a],
                        out_ref.at[pl.ds(cid * NPAD + sid * 640, 640)])

    @pl.when(sid == 15)
    def _():
        pltpu.sync_copy(deg_sh.at[sl_b],
                        out_ref.at[pl.ds(cid * NPAD + 15 * 640, 512)])


@functools.partial(
    pl.kernel,
    out_type=jax.ShapeDtypeStruct((2, NPAD, D), jnp.float32),
    mesh=_mesh,
    scratch_types=(
        [pltpu.VMEM((CH2,), jnp.int32) for _ in range(2 * NBUF)]
        + [pltpu.VMEM((CH2, D), jnp.float32) for _ in range(NBUF)]
        + [pltpu.VMEM_SHARED((NPAD, D), jnp.float32)]
        + [pltpu.SemaphoreType.DMA for _ in range(4 * NBUF + 1)]
    ),
)
def _hop_kernel(row_ref, col_ref, g_ref, z2_ref, out_ref, *scr):
    ridx = scr[0:NBUF]
    cidx = scr[NBUF:2 * NBUF]
    rows = scr[2 * NBUF:3 * NBUF]
    acc_sh = scr[3 * NBUF]
    gir = scr[3 * NBUF + 1:3 * NBUF + 1 + NBUF]
    gic = scr[3 * NBUF + 1 + NBUF:3 * NBUF + 1 + 2 * NBUF]
    gg = scr[3 * NBUF + 1 + 2 * NBUF:3 * NBUF + 1 + 3 * NBUF]
    gs = scr[3 * NBUF + 1 + 3 * NBUF:3 * NBUF + 1 + 4 * NBUF]
    zs = scr[3 * NBUF + 1 + 4 * NBUF]
    cid = lax.axis_index("c")
    sid = lax.axis_index("s")
    wid = cid * 16 + sid
    sl = pl.ds(sid * ROWS_PER_TILE, ROWS_PER_TILE)
    pltpu.async_copy(z2_ref.at[sl], acc_sh.at[sl], zs)

    for p in range(NBUF):
        pltpu.async_copy(row_ref.at[p, wid, 0], ridx[p], gir[p])
        pltpu.async_copy(col_ref.at[p, wid, 0], cidx[p], gic[p])
    for p in range(NBUF):
        pltpu.make_async_copy(row_ref.at[p, wid, 0], ridx[p], gir[p]).wait()
        pltpu.async_copy(g_ref.at[ridx[p]], rows[p], gg[p])
    pltpu.make_async_copy(z2_ref.at[sl], acc_sh.at[sl], zs).wait()
    plsc.subcore_barrier()

    def body(i, carry):
        c = NBUF * i
        for p in range(NBUF):
            pltpu.make_async_copy(g_ref.at[ridx[p]], rows[p], gg[p]).wait()
            pltpu.make_async_copy(
                col_ref.at[c + p, wid, 0], cidx[p], gic[p]).wait()
            pltpu.async_copy(rows[p], acc_sh.at[cidx[p]], gs[p], add=True)
            pltpu.async_copy(
                row_ref.at[c + p + NBUF, wid, 0], ridx[p], gir[p])
        for p in range(NBUF):
            pltpu.make_async_copy(rows[p], acc_sh.at[cidx[p]], gs[p]).wait()
            pltpu.async_copy(
                col_ref.at[c + p + NBUF, wid, 0], cidx[p], gic[p])
            pltpu.make_async_copy(
                row_ref.at[c + p + NBUF, wid, 0], ridx[p], gir[p]).wait()
            pltpu.async_copy(g_ref.at[ridx[p]], rows[p], gg[p])
        return carry

    lax.fori_loop(0, NCH2 // NBUF - 1, body, 0)
    ce = NCH2 - NBUF
    for p in range(NBUF):
        pltpu.make_async_copy(g_ref.at[ridx[p]], rows[p], gg[p]).wait()
        pltpu.make_async_copy(
            col_ref.at[ce + p, wid, 0], cidx[p], gic[p]).wait()
        pltpu.async_copy(rows[p], acc_sh.at[cidx[p]], gs[p], add=True)
    for p in range(NBUF):
        pltpu.make_async_copy(rows[p], acc_sh.at[cidx[p]], gs[p]).wait()
    plsc.subcore_barrier()
    pltpu.sync_copy(acc_sh.at[sl], out_ref.at[cid, sl])


_BN = 632
_GRID = NPAD // _BN
NREAL = 10000


def _tc1_body(d_ref, x_ref, o_ref):
    cnt = d_ref[:, 0] + d_ref[:, 1] + 1.0
    dis = lax.rsqrt(cnt)
    r = pl.program_id(0) * _BN + lax.broadcasted_iota(jnp.int32, (_BN, 1), 0)
    o_ref[...] = jnp.where(r < NREAL, x_ref[...] * dis[:, None], 0.0)


def _tc2_body(d_ref, s_ref, g_ref, o_ref):
    cnt = d_ref[:, 0] + d_ref[:, 1] + 1.0
    inv = 1.0 / cnt
    r = pl.program_id(0) * _BN + lax.broadcasted_iota(jnp.int32, (_BN, 1), 0)
    o_ref[...] = jnp.where(
        r < NREAL, (s_ref[0] + s_ref[1] + g_ref[...]) * inv[:, None], 0.0)


def _tc3_body(d_ref, x_ref, g2_ref, s2_ref, w0_ref, b0_ref, w1_ref, b1_ref,
              w2_ref, b2_ref, wf_ref, bf_ref, o_ref):
    cnt = d_ref[:, 0] + d_ref[:, 1] + 1.0
    dis = lax.rsqrt(cnt)
    sq = jnp.sqrt(cnt)
    g2 = g2_ref[...]
    x1 = g2 * sq[:, None]
    x2 = (s2_ref[0] + s2_ref[1] + g2) * dis[:, None]
    dot = functools.partial(jnp.dot, preferred_element_type=jnp.float32)
    t0 = jnp.maximum(dot(x_ref[...], w0_ref[...]) + b0_ref[...], 0.0)
    t1 = jnp.maximum(dot(x1, w1_ref[...]) + b1_ref[...], 0.0)
    t2 = jnp.maximum(dot(x2, w2_ref[...]) + b2_ref[...], 0.0)
    wf = wf_ref[...]
    out = dot(t0, wf[:64]) + dot(t1, wf[64:128]) + dot(t2, wf[128:]) \
        + bf_ref[...]
    o_ref[...] = out


def _spec_d():
    return pl.BlockSpec((_BN, 2), lambda i: (i, 0))


def _spec_rows():
    return pl.BlockSpec((_BN, D), lambda i: (i, 0))


def _spec_pair():
    return pl.BlockSpec((2, _BN, D), lambda i: (0, i, 0))


def _spec_full(shape):
    return pl.BlockSpec(shape, lambda i: tuple(0 for _ in shape))


def kernel(x, edge_index, W0, b0, W1, b1, W2, b2, Wf, bf):
    n, d = x.shape
    f32 = jnp.float32

    e = edge_index.shape[1]
    i32 = jnp.int32
    pidx = jnp.arange(EPAD - e, dtype=i32)
    row_rs = jnp.concatenate(
        [edge_index[0], n + pidx % (NPAD - n)]).reshape(NCH2, 32, 1, CH2)
    col_rs = jnp.concatenate(
        [edge_index[1], pidx % NPAD]).reshape(NCH2, 32, 1, CH2)
    pidx2 = jnp.arange(32 * NCH * CH - e, dtype=i32)
    col_dg = jnp.concatenate(
        [edge_index[1], n + pidx2 % (NPAD - n)]).reshape(32, NCH, CH)
    z1 = jnp.zeros((NPAD,), f32)
    z2 = jnp.zeros((NPAD, D), f32)
    ones = jnp.ones((CH,), f32)

    deg2 = _deg_kernel(col_dg, ones, z1).reshape(2, NPAD).T

    g1 = pl.pallas_call(
        _tc1_body,
        grid=(_GRID,),
        in_specs=[_spec_d(), _spec_rows()],
        out_specs=_spec_rows(),
        out_shape=jax.ShapeDtypeStruct((NPAD, D), f32),
    )(deg2, x)

    s1p = _hop_kernel(row_rs, col_rs, g1, z2)

    g2 = pl.pallas_call(
        _tc2_body,
        grid=(_GRID,),
        in_specs=[_spec_d(), _spec_pair(), _spec_rows()],
        out_specs=_spec_rows(),
        out_shape=jax.ShapeDtypeStruct((NPAD, D), f32),
    )(deg2, s1p, g1)

    s2p = _hop_kernel(row_rs, col_rs, g2, z2)

    out = pl.pallas_call(
        _tc3_body,
        grid=(_GRID,),
        in_specs=[
            _spec_d(), _spec_rows(), _spec_rows(), _spec_pair(),
            _spec_full((D, 64)), _spec_full((1, 64)),
            _spec_full((D, 64)), _spec_full((1, 64)),
            _spec_full((D, 64)), _spec_full((1, 64)),
            _spec_full((192, D)), _spec_full((1, D)),
        ],
        out_specs=_spec_rows(),
        out_shape=jax.ShapeDtypeStruct((n, D), f32),
    )(deg2, x, g2, s2p, W0, b0.reshape(1, 64), W1, b1.reshape(1, 64),
      W2, b2.reshape(1, 64), Wf, bf.reshape(1, D))

    return out
